# baseline (speedup 1.0000x reference)
import jax
import jax.numpy as jnp
from jax import lax
from jax.experimental import pallas as pl
from jax.experimental.pallas import tpu as pltpu
from jax.experimental.pallas import tpu_sc as plsc

_NC = 2
_NS = 16
_NW = _NC * _NS

_N_IDX = 4096 * 200
_PER_W = _N_IDX // _NW
_CHUNK = 256
_N_CHUNK = _PER_W // _CHUNK


def _gather_body(idx_hbm, tab_hbm, out_hbm,
                 ibuf, rbuf, isem0, isem1, gsem0, gsem1):
    wid = lax.axis_index("s") * _NC + lax.axis_index("c")
    base = wid * _PER_W
    isems = (isem0, isem1)
    gsems = (gsem0, gsem1)

    def istart(j, b):
        pltpu.async_copy(
            idx_hbm.at[pl.ds(base + j * _CHUNK, _CHUNK)], ibuf.at[b], isems[b]
        )

    def iwait(b):
        pltpu.make_async_copy(
            idx_hbm.at[pl.ds(0, _CHUNK)], ibuf.at[b], isems[b]
        ).wait()

    def gstart(b):
        pltpu.async_copy(tab_hbm.at[ibuf.at[b]], rbuf.at[b], gsems[b])

    def gwait(b):
        pltpu.make_async_copy(
            tab_hbm.at[ibuf.at[b]], rbuf.at[b], gsems[b]
        ).wait()

    istart(0, 0)
    istart(1, 1)
    iwait(0)
    gstart(0)

    @pl.loop(0, _N_CHUNK, step=2)
    def _grp(j0):
        for b in range(2):
            j = j0 + b
            nb = 1 - b

            @pl.when(j + 1 < _N_CHUNK)
            def _():
                iwait(nb)
                gstart(nb)

            gwait(b)

            @pl.when(j + 2 < _N_CHUNK)
            def _():
                istart(j + 2, b)

            pltpu.sync_copy(
                rbuf.at[b], out_hbm.at[pl.ds(base + j * _CHUNK, _CHUNK)]
            )


def kernel(control_tokens, embedding_table):
    b, h = control_tokens.shape
    _, d = embedding_table.shape
    idx = control_tokens.reshape(-1).astype(jnp.int32)

    mesh = plsc.VectorSubcoreMesh(
        core_axis_name="c", subcore_axis_name="s",
        num_cores=_NC, num_subcores=_NS,
    )
    run = pl.kernel(
        _gather_body,
        out_type=jax.ShapeDtypeStruct((_N_IDX, d), jnp.float32),
        mesh=mesh,
        scratch_types=[
            pltpu.VMEM((2, _CHUNK), jnp.int32),
            pltpu.VMEM((2, _CHUNK, d), jnp.float32),
            pltpu.SemaphoreType.DMA,
            pltpu.SemaphoreType.DMA,
            pltpu.SemaphoreType.DMA,
            pltpu.SemaphoreType.DMA,
        ],
        compiler_params=pltpu.CompilerParams(use_tc_tiling_on_sc=False),
    )
    out = run(idx, embedding_table)
    return out.reshape(b, h, d)


# speedup vs baseline: 1.0263x; 1.0263x over previous
import jax
import jax.numpy as jnp
from jax import lax
from jax.experimental import pallas as pl
from jax.experimental.pallas import tpu as pltpu
from jax.experimental.pallas import tpu_sc as plsc

_NC = 2
_NS = 16
_NW = _NC * _NS

_N_IDX = 4096 * 200
_PER_W = _N_IDX // _NW
_CHUNK = 512
_N_CHUNK = _PER_W // _CHUNK


_NBUF = 5


def _gather_body(idx_hbm, tab_hbm, out_hbm, idx_v, rows_v, *gsems):
    wid = lax.axis_index("s") * _NC + lax.axis_index("c")
    base = wid * _PER_W
    pltpu.sync_copy(idx_hbm.at[pl.ds(base, _PER_W)], idx_v)

    def start_gather(j, b):
        pltpu.async_copy(
            tab_hbm.at[idx_v.at[pl.ds(j * _CHUNK, _CHUNK)]],
            rows_v.at[b], gsems[b],
        )

    for j in range(_NBUF - 1):
        start_gather(j, j)

    @pl.loop(0, _N_CHUNK, step=_NBUF)
    def _grp(j0):
        for b in range(_NBUF):
            j = j0 + b
            nb = (b + _NBUF - 1) % _NBUF

            @pl.when(j + _NBUF - 1 < _N_CHUNK)
            def _():
                start_gather(j + _NBUF - 1, nb)

            pltpu.make_async_copy(
                tab_hbm.at[idx_v.at[pl.ds(0, _CHUNK)]],
                rows_v.at[b], gsems[b],
            ).wait()
            pltpu.sync_copy(
                rows_v.at[b], out_hbm.at[pl.ds(base + j * _CHUNK, _CHUNK)]
            )


def kernel(control_tokens, embedding_table):
    b, h = control_tokens.shape
    _, d = embedding_table.shape
    idx = control_tokens.reshape(-1).astype(jnp.int32)

    mesh = plsc.VectorSubcoreMesh(
        core_axis_name="c", subcore_axis_name="s",
        num_cores=_NC, num_subcores=_NS,
    )
    run = pl.kernel(
        _gather_body,
        out_type=jax.ShapeDtypeStruct((_N_IDX, d), jnp.float32),
        mesh=mesh,
        scratch_types=[
            pltpu.VMEM((_PER_W,), jnp.int32),
            pltpu.VMEM((_NBUF, _CHUNK, d), jnp.float32),
        ] + [pltpu.SemaphoreType.DMA] * _NBUF,
        compiler_params=pltpu.CompilerParams(use_tc_tiling_on_sc=False),
    )
    out = run(idx, embedding_table)
    return out.reshape(b, h, d)


# speedup vs baseline: 1.0265x; 1.0002x over previous
import jax
import jax.numpy as jnp
from jax import lax
from jax.experimental import pallas as pl
from jax.experimental.pallas import tpu as pltpu
from jax.experimental.pallas import tpu_sc as plsc

_NC = 2
_NS = 16
_NW = _NC * _NS

_N_IDX = 4096 * 200
_PER_W = _N_IDX // _NW
_CHUNK = 512
_N_CHUNK = _PER_W // _CHUNK


_NBUF = 5


def _gather_body(idx_hbm, tab_hbm, out_hbm, idx_v, rows_v, *sems):
    gsems = sems[:_NBUF]
    wsems = sems[_NBUF:]
    wid = lax.axis_index("s") * _NC + lax.axis_index("c")
    base = wid * _PER_W
    pltpu.sync_copy(idx_hbm.at[pl.ds(base, _PER_W)], idx_v)

    def start_gather(j, b):
        pltpu.async_copy(
            tab_hbm.at[idx_v.at[pl.ds(j * _CHUNK, _CHUNK)]],
            rows_v.at[b], gsems[b],
        )

    def wait_write(b):
        pltpu.make_async_copy(
            rows_v.at[b], out_hbm.at[pl.ds(base, _CHUNK)], wsems[b]
        ).wait()

    for j in range(_NBUF - 1):
        start_gather(j, j)

    @pl.loop(0, _N_CHUNK, step=_NBUF)
    def _grp(j0):
        for b in range(_NBUF):
            j = j0 + b
            nb = (b + _NBUF - 1) % _NBUF

            @pl.when(j + _NBUF - 1 < _N_CHUNK)
            def _():

                @pl.when(j >= 1)
                def _():
                    wait_write(nb)

                start_gather(j + _NBUF - 1, nb)

            pltpu.make_async_copy(
                tab_hbm.at[idx_v.at[pl.ds(0, _CHUNK)]],
                rows_v.at[b], gsems[b],
            ).wait()
            pltpu.async_copy(
                rows_v.at[b], out_hbm.at[pl.ds(base + j * _CHUNK, _CHUNK)],
                wsems[b],
            )

    for j in range(_N_CHUNK - _NBUF, _N_CHUNK):
        wait_write(j % _NBUF)


def kernel(control_tokens, embedding_table):
    b, h = control_tokens.shape
    _, d = embedding_table.shape
    idx = control_tokens.reshape(-1).astype(jnp.int32)

    mesh = plsc.VectorSubcoreMesh(
        core_axis_name="c", subcore_axis_name="s",
        num_cores=_NC, num_subcores=_NS,
    )
    run = pl.kernel(
        _gather_body,
        out_type=jax.ShapeDtypeStruct((_N_IDX, d), jnp.float32),
        mesh=mesh,
        scratch_types=[
            pltpu.VMEM((_PER_W,), jnp.int32),
            pltpu.VMEM((_NBUF, _CHUNK, d), jnp.float32),
        ] + [pltpu.SemaphoreType.DMA] * (2 * _NBUF),
        compiler_params=pltpu.CompilerParams(use_tc_tiling_on_sc=False),
    )
    out = run(idx, embedding_table)
    return out.reshape(b, h, d)


# speedup vs baseline: 1.4033x; 1.3671x over previous
import jax
import jax.numpy as jnp
from jax import lax
from jax.experimental import pallas as pl
from jax.experimental.pallas import tpu as pltpu
from jax.experimental.pallas import tpu_sc as plsc

_NC = 2
_NS = 16
_NW = _NC * _NS

_N_IDX = 4096 * 200
_PER_W = _N_IDX // _NW
_CHUNK = 512
_N_CHUNK = _PER_W // _CHUNK


_NBUF = 5


def _gather_body(idx_hbm, tab_hbm, out_hbm, idx_v, rows_v, *sems):
    gsems = sems[:_NBUF]
    wsems = sems[_NBUF:]
    wid = lax.axis_index("s") * _NC + lax.axis_index("c")
    base = wid * _PER_W
    pltpu.sync_copy(idx_hbm.at[pl.ds(base, _PER_W)], idx_v)

    def start_gather(j, b):
        pltpu.async_copy(
            tab_hbm.at[idx_v.at[pl.ds(j * _CHUNK, _CHUNK)]],
            rows_v.at[b], gsems[b],
        )

    def wait_write(b):
        pltpu.make_async_copy(
            rows_v.at[b], out_hbm.at[pl.ds(base, _CHUNK), pl.ds(0, 32)], wsems[b]
        ).wait()

    for j in range(_NBUF - 1):
        start_gather(j, j)

    @pl.loop(0, _N_CHUNK, step=_NBUF)
    def _grp(j0):
        for b in range(_NBUF):
            j = j0 + b
            nb = (b + _NBUF - 1) % _NBUF

            @pl.when(j + _NBUF - 1 < _N_CHUNK)
            def _():

                @pl.when(j >= 1)
                def _():
                    wait_write(nb)

                start_gather(j + _NBUF - 1, nb)

            pltpu.make_async_copy(
                tab_hbm.at[idx_v.at[pl.ds(0, _CHUNK)]],
                rows_v.at[b], gsems[b],
            ).wait()
            pltpu.async_copy(
                rows_v.at[b],
                out_hbm.at[pl.ds(base + j * _CHUNK, _CHUNK), pl.ds(0, 32)],
                wsems[b],
            )

    for j in range(_N_CHUNK - _NBUF, _N_CHUNK):
        wait_write(j % _NBUF)


def kernel(control_tokens, embedding_table):
    b, h = control_tokens.shape
    _, d = embedding_table.shape
    idx = control_tokens.reshape(-1).astype(jnp.int32)

    mesh = plsc.VectorSubcoreMesh(
        core_axis_name="c", subcore_axis_name="s",
        num_cores=_NC, num_subcores=_NS,
    )
    run = pl.kernel(
        _gather_body,
        out_type=jax.ShapeDtypeStruct((_N_IDX, 128), jnp.float32),
        mesh=mesh,
        scratch_types=[
            pltpu.VMEM((_PER_W,), jnp.int32),
            pltpu.VMEM((_NBUF, _CHUNK, d), jnp.float32),
        ] + [pltpu.SemaphoreType.DMA] * (2 * _NBUF),
        compiler_params=pltpu.CompilerParams(use_tc_tiling_on_sc=False),
    )
    out = run(idx, embedding_table)
    return out[:, :d].reshape(b, h, d)
